# baseline (speedup 1.0000x reference)
import functools

import jax
import jax.numpy as jnp
from jax import lax
from jax.experimental import pallas as pl
from jax.experimental.pallas import tpu as pltpu
from jax.experimental.pallas import tpu_sc as plsc

N = 40000
C = 3
K = 256
Q_MIN = 0.1
S_B = 1.0
EPS = 0.001
BMAX = 1.0 - 1e-4

NSC = 16
CS = 2560
CS_LAST = N - CS * (NSC - 1)
NV = CS // 16
NV_LAST = CS_LAST // 16
KS = K // NSC
BIGI = 2 ** 30

BN = 2048
NB = 20

NPAD = 40960


def _qp_body(beta_ref, q_ref, p_ref):
    b = jnp.clip(beta_ref[...], 0.0, BMAX)
    x = b * (1.0 - EPS)
    at = 0.5 * jnp.log((1.0 + x) / (1.0 - x))
    p = at * at
    p_ref[...] = p
    q_ref[...] = p + Q_MIN


def _qp_call(beta_r):
    return pl.pallas_call(
        _qp_body,
        out_shape=(
            jax.ShapeDtypeStruct((N,), jnp.float32),
            jax.ShapeDtypeStruct((N,), jnp.float32),
        ),
    )(beta_r)


def _seg_body(beta_hbm, p_hbm, asso_hbm,
              key_out, idx_out, cnt_out, psum_out, pl_out,
              beta_buf, p_buf, asso_buf, plo_buf,
              key_tab, idx_tab, cnt_tab, psum_tab,
              lkey, lidx, lcnt, lpsum,
              mkey, midx, mcnt, mpsum,
              okey, oidx, ocnt, opsum,
              psum_all, cnt_all,
              sh_key, sh_idx, sh_cnt, sh_psum,
              sh_psum_m, sh_cnt_m):
    wid = lax.axis_index("s")
    start = wid * CS
    lane = lax.iota(jnp.int32, 16)

    @pl.when(wid < NSC - 1)
    def _():
        pltpu.sync_copy(beta_hbm.at[pl.ds(start, CS)], beta_buf)
        pltpu.sync_copy(p_hbm.at[pl.ds(start, CS)], p_buf)
        pltpu.sync_copy(asso_hbm.at[pl.ds(start, CS)], asso_buf)

    @pl.when(wid == NSC - 1)
    def _():
        pltpu.sync_copy(beta_hbm.at[pl.ds(CS * (NSC - 1), CS_LAST)],
                        beta_buf.at[pl.ds(0, CS_LAST)])
        pltpu.sync_copy(p_hbm.at[pl.ds(CS * (NSC - 1), CS_LAST)],
                        p_buf.at[pl.ds(0, CS_LAST)])
        pltpu.sync_copy(asso_hbm.at[pl.ds(CS * (NSC - 1), CS_LAST)],
                        asso_buf.at[pl.ds(0, CS_LAST)])

    nv = jnp.where(wid < NSC - 1, NV, NV_LAST)

    def init_body(i, _):
        sl = pl.ds(i * 16, 16)
        key_tab[sl] = jnp.full((16,), -1.0, jnp.float32)
        idx_tab[sl] = jnp.zeros((16,), jnp.int32)
        cnt_tab[sl] = jnp.zeros((16,), jnp.float32)
        psum_tab[sl] = jnp.zeros((16,), jnp.float32)
        return 0

    lax.fori_loop(0, K, init_body, 0)

    def scat_body(it, _):
        sl = pl.ds(it * 16, 16)
        bv = jnp.clip(beta_buf[sl], 0.0, BMAX)
        pv = p_buf[sl]
        av = asso_buf[sl]
        gidx = start + it * 16 + lane
        addr = lane * K + av
        cur = plsc.load_gather(key_tab, [addr])
        m = bv > cur
        plsc.store_scatter(key_tab, [addr], bv, mask=m)
        plsc.store_scatter(idx_tab, [addr], gidx, mask=m)
        plsc.addupdate_scatter(cnt_tab, [addr], jnp.ones((16,), jnp.float32))
        plsc.addupdate_scatter(psum_tab, [addr],
                               jnp.maximum(pv, 1e-6))
        return 0

    lax.fori_loop(0, nv, scat_body, 0)

    def red_body(g, _):
        k0 = g * 16
        mx = jnp.full((16,), -1.0, jnp.float32)
        sc = jnp.zeros((16,), jnp.float32)
        sp = jnp.zeros((16,), jnp.float32)
        cks = []
        for l in range(16):
            sl = pl.ds(k0 + l * K, 16)
            ck = key_tab[sl]
            cks.append(ck)
            mx = jnp.maximum(mx, ck)
            sc = sc + cnt_tab[sl]
            sp = sp + psum_tab[sl]
        mn = jnp.full((16,), BIGI, jnp.int32)
        for l in range(16):
            ci = idx_tab[pl.ds(k0 + l * K, 16)]
            mn = jnp.minimum(mn, jnp.where(cks[l] == mx, ci, BIGI))
        osl = pl.ds(k0, 16)
        lkey[osl] = mx
        lidx[osl] = mn
        lcnt[osl] = sc
        lpsum[osl] = sp
        return 0

    lax.fori_loop(0, K // 16, red_body, 0)

    pltpu.sync_copy(lkey, sh_key.at[pl.ds(wid * K, K)])
    pltpu.sync_copy(lidx, sh_idx.at[pl.ds(wid * K, K)])
    pltpu.sync_copy(lcnt, sh_cnt.at[pl.ds(wid * K, K)])
    pltpu.sync_copy(lpsum, sh_psum.at[pl.ds(wid * K, K)])
    plsc.subcore_barrier()

    k0 = wid * KS
    pltpu.sync_copy(sh_key, mkey)
    pltpu.sync_copy(sh_idx, midx)
    pltpu.sync_copy(sh_cnt, mcnt)
    pltpu.sync_copy(sh_psum, mpsum)

    mx = jnp.full((KS,), -1.0, jnp.float32)
    sc = jnp.zeros((KS,), jnp.float32)
    sp = jnp.zeros((KS,), jnp.float32)
    rks = []
    for w in range(NSC):
        sl = pl.ds(w * K + k0, KS)
        rk = mkey[sl]
        rks.append(rk)
        mx = jnp.maximum(mx, rk)
        sc = sc + mcnt[sl]
        sp = sp + mpsum[sl]
    mn = jnp.full((KS,), BIGI, jnp.int32)
    for w in range(NSC):
        mn = jnp.minimum(
            mn, jnp.where(rks[w] == mx, midx[pl.ds(w * K + k0, KS)], BIGI))
    okey[...] = mx
    oidx[...] = mn
    ocnt[...] = sc
    opsum[...] = sp

    pltpu.sync_copy(okey, key_out.at[pl.ds(k0, KS)])
    pltpu.sync_copy(oidx, idx_out.at[pl.ds(k0, KS)])
    pltpu.sync_copy(ocnt, cnt_out.at[pl.ds(k0, KS)])
    pltpu.sync_copy(opsum, psum_out.at[pl.ds(k0, KS)])
    pltpu.sync_copy(opsum, sh_psum_m.at[pl.ds(k0, KS)])
    pltpu.sync_copy(ocnt, sh_cnt_m.at[pl.ds(k0, KS)])
    plsc.subcore_barrier()

    pltpu.sync_copy(sh_psum_m, psum_all)
    pltpu.sync_copy(sh_cnt_m, cnt_all)

    kexist = jnp.float32(0.0)
    for i in range(K // 16):
        cv = cnt_all[pl.ds(i * 16, 16)]
        kexist = kexist + jnp.sum(jnp.where(cv > 0.0, 1.0, 0.0))
    ktot = jnp.maximum(kexist, 1.0)

    def pls_body(it, _):
        sl = pl.ds(it * 16, 16)
        av = asso_buf[sl]
        ps = plsc.load_gather(psum_all, [av])
        pm = jnp.maximum(p_buf[sl], 1e-6)
        plo_buf[sl] = pm / (jnp.maximum(ps, 1e-9) * ktot)
        return 0

    lax.fori_loop(0, nv, pls_body, 0)

    @pl.when(wid < NSC - 1)
    def _():
        pltpu.sync_copy(plo_buf, pl_out.at[pl.ds(start, CS)])

    @pl.when(wid == NSC - 1)
    def _():
        pltpu.sync_copy(plo_buf.at[pl.ds(0, CS_LAST)],
                        pl_out.at[pl.ds(CS * (NSC - 1), CS_LAST)])


def _seg_call(beta_r, p_r, asso_r):
    mesh = plsc.VectorSubcoreMesh(
        core_axis_name="c", subcore_axis_name="s", num_cores=1)
    f = pl.kernel(
        _seg_body,
        out_type=(
            jax.ShapeDtypeStruct((K,), jnp.float32),
            jax.ShapeDtypeStruct((K,), jnp.int32),
            jax.ShapeDtypeStruct((K,), jnp.float32),
            jax.ShapeDtypeStruct((K,), jnp.float32),
            jax.ShapeDtypeStruct((N,), jnp.float32),
        ),
        mesh=mesh,
        compiler_params=pltpu.CompilerParams(needs_layout_passes=False),
        scratch_types=[
            pltpu.VMEM((CS,), jnp.float32),
            pltpu.VMEM((CS,), jnp.float32),
            pltpu.VMEM((CS,), jnp.int32),
            pltpu.VMEM((CS,), jnp.float32),
            pltpu.VMEM((K * 16,), jnp.float32),
            pltpu.VMEM((K * 16,), jnp.int32),
            pltpu.VMEM((K * 16,), jnp.float32),
            pltpu.VMEM((K * 16,), jnp.float32),
            pltpu.VMEM((K,), jnp.float32),
            pltpu.VMEM((K,), jnp.int32),
            pltpu.VMEM((K,), jnp.float32),
            pltpu.VMEM((K,), jnp.float32),
            pltpu.VMEM((NSC * K,), jnp.float32),
            pltpu.VMEM((NSC * K,), jnp.int32),
            pltpu.VMEM((NSC * K,), jnp.float32),
            pltpu.VMEM((NSC * K,), jnp.float32),
            pltpu.VMEM((KS,), jnp.float32),
            pltpu.VMEM((KS,), jnp.int32),
            pltpu.VMEM((KS,), jnp.float32),
            pltpu.VMEM((KS,), jnp.float32),
            pltpu.VMEM((K,), jnp.float32),
            pltpu.VMEM((K,), jnp.float32),
            pltpu.VMEM_SHARED((NSC * K,), jnp.float32),
            pltpu.VMEM_SHARED((NSC * K,), jnp.int32),
            pltpu.VMEM_SHARED((NSC * K,), jnp.float32),
            pltpu.VMEM_SHARED((NSC * K,), jnp.float32),
            pltpu.VMEM_SHARED((K,), jnp.float32),
            pltpu.VMEM_SHARED((K,), jnp.float32),
        ],
    )
    return f(beta_r, p_r, asso_r)


ROWS = 16
NDB = NPAD // (128 * ROWS)


def _dense_body(coords_ref, q_ref, asso_ref, xa_ref, balpha_ref, cnt_ref,
                lv_ref, lr_ref, lb_ref, pp_ref,
                ca_s, cr_s, amat_s, acc_a, acc_r):
    i = pl.program_id(0)

    @pl.when(i == 0)
    def _():
        ba = balpha_ref[...]
        cnt = cnt_ref[...]
        x = ba * (1.0 - EPS)
        at = 0.5 * jnp.log((1.0 + x) / (1.0 - x))
        qa = at * at + Q_MIN
        exists = jnp.where(cnt > 0.0, 1.0, 0.0)
        ktot = jnp.maximum(jnp.sum(exists), 1.0)
        nk = jnp.maximum(cnt, 1.0)
        nrep = jnp.maximum(jnp.float32(N) - cnt, 1.0)
        ca_s[...] = qa * exists / (nk * ktot)
        cr_s[...] = qa * exists / (nrep * ktot)
        xa = xa_ref[...]
        xa2c = jnp.sum(xa * xa, axis=1, keepdims=True)
        amat_s[...] = jnp.concatenate(
            [-2.0 * xa, xa2c, jnp.ones((K, 1), jnp.float32)], axis=1)
        acc_a[...] = jnp.zeros((1, 128), jnp.float32)
        acc_r[...] = jnp.zeros((1, 128), jnp.float32)

    kio = lax.broadcasted_iota(jnp.int32, (K, 128), 0)
    ca = ca_s[...]
    cr = cr_s[...]
    amat = amat_s[...]
    qb = q_ref[...].reshape(ROWS, 128)
    ab = asso_ref[...].reshape(ROWS, 128)
    lanei = lax.broadcasted_iota(jnp.int32, (1, 128), 1)

    aa = acc_a[...]
    rr = acc_r[...]
    for r in range(ROWS):
        xr3 = coords_ref[r * 128:(r + 1) * 128, :]
        x2c = jnp.sum(xr3 * xr3, axis=1, keepdims=True)
        xr5 = jnp.concatenate(
            [xr3, jnp.ones((128, 1), jnp.float32), x2c], axis=1)
        d2 = jnp.maximum(
            lax.dot_general(amat, xr5, (((1,), (1,)), ((), ())),
                            preferred_element_type=jnp.float32), 0.0)
        z = d2 + 1e-9
        hinge = jnp.maximum(0.0, 1.0 - z * lax.rsqrt(z))

        asso_row = ab[r:r + 1, :]
        q_row = qb[r:r + 1, :]
        member = asso_row == kio
        attr = jnp.where(member, d2, 0.0)
        rep = jnp.where(member, 0.0, hinge)

        rows_a = jnp.dot(ca, attr, preferred_element_type=jnp.float32)
        rows_r = jnp.dot(cr, rep, preferred_element_type=jnp.float32)
        valid = (i * (ROWS * 128) + r * 128 + lanei) < N
        va = jnp.where(valid, q_row * rows_a, 0.0)
        vr = jnp.where(valid, q_row * rows_r, 0.0)
        pp_ref[:, r:r + 1, :] = (va + vr).reshape(1, 1, 128)
        aa = aa + va
        rr = rr + vr
    acc_a[...] = aa
    acc_r[...] = rr

    @pl.when(i == NDB - 1)
    def _():
        lv_ref[...] = jnp.full((1, 128), jnp.sum(aa), jnp.float32)
        lr_ref[...] = jnp.full((1, 128), jnp.sum(rr), jnp.float32)
        cnt = cnt_ref[...]
        exists = jnp.where(cnt > 0.0, 1.0, 0.0)
        ktot = jnp.maximum(jnp.sum(exists), 1.0)
        lb = jnp.sum((1.0 - balpha_ref[...]) * exists) / ktot
        lb_ref[...] = jnp.full((1, 128), lb, jnp.float32)


def _dense_call(coords, q_r, asso_r, x_alpha, balpha2, cnt2):
    return pl.pallas_call(
        _dense_body,
        grid=(NDB,),
        in_specs=[
            pl.BlockSpec((ROWS * 128, C), lambda i: (i, 0)),
            pl.BlockSpec((ROWS * 128,), lambda i: (i,)),
            pl.BlockSpec((ROWS * 128,), lambda i: (i,)),
            pl.BlockSpec((K, C), lambda i: (0, 0)),
            pl.BlockSpec((1, K), lambda i: (0, 0)),
            pl.BlockSpec((1, K), lambda i: (0, 0)),
        ],
        out_specs=[
            pl.BlockSpec((1, 128), lambda i: (0, 0)),
            pl.BlockSpec((1, 128), lambda i: (0, 0)),
            pl.BlockSpec((1, 128), lambda i: (0, 0)),
            pl.BlockSpec((1, ROWS, 128), lambda i: (i, 0, 0)),
        ],
        out_shape=[
            jax.ShapeDtypeStruct((1, 128), jnp.float32),
            jax.ShapeDtypeStruct((1, 128), jnp.float32),
            jax.ShapeDtypeStruct((1, 128), jnp.float32),
            jax.ShapeDtypeStruct((NDB, ROWS, 128), jnp.float32),
        ],
        scratch_shapes=[
            pltpu.VMEM((1, K), jnp.float32),
            pltpu.VMEM((1, K), jnp.float32),
            pltpu.VMEM((K, C + 2), jnp.float32),
            pltpu.VMEM((1, 128), jnp.float32),
            pltpu.VMEM((1, 128), jnp.float32),
        ],
    )(coords, q_r, asso_r, x_alpha, balpha2, cnt2)


@jax.jit
def _run(beta, coords, asso_idx):
    beta_r = beta.reshape(-1)
    q_r, p_r = _qp_call(beta_r)

    asso_r = asso_idx.reshape(-1)
    balpha, aidx, cnt, psum, pl_sc = _seg_call(beta_r, p_r, asso_r)
    del psum

    x_alpha = coords[aidx]

    lv, lr, lb, pp_pad = _dense_call(
        coords, q_r, asso_r,
        x_alpha, balpha.reshape(1, K), cnt.reshape(1, K))

    per_pt = pp_pad.reshape(-1)[:N].reshape(N, 1)
    return (lv[0, 0], lr[0, 0], lb[0, 0],
            pl_sc.reshape(N, 1), per_pt)


def kernel(beta, coords, asso_idx, row_splits):
    del row_splits
    return _run(beta, coords, asso_idx)


# speedup vs baseline: 1.0714x; 1.0714x over previous
import functools

import jax
import jax.numpy as jnp
from jax import lax
from jax.experimental import pallas as pl
from jax.experimental.pallas import tpu as pltpu
from jax.experimental.pallas import tpu_sc as plsc

N = 40000
C = 3
K = 256
Q_MIN = 0.1
S_B = 1.0
EPS = 0.001
BMAX = 1.0 - 1e-4

NSC = 16
CS = 2560
CS_LAST = N - CS * (NSC - 1)
NV = CS // 16
NV_LAST = CS_LAST // 16
KS = K // NSC
BIGI = 2 ** 30

BN = 2048
NB = 20

NPAD = 40960


def _qp_body(beta_ref, q_ref, p_ref):
    b = jnp.clip(beta_ref[...], 0.0, BMAX)
    x = b * (1.0 - EPS)
    at = 0.5 * jnp.log((1.0 + x) / (1.0 - x))
    p = at * at
    p_ref[...] = p
    q_ref[...] = p + Q_MIN


def _qp_call(beta_pad):
    return pl.pallas_call(
        _qp_body,
        out_shape=(
            jax.ShapeDtypeStruct((NPAD // 128, 128), jnp.float32),
            jax.ShapeDtypeStruct((NPAD // 128, 128), jnp.float32),
        ),
    )(beta_pad)


def _seg_body(beta_hbm, p_hbm, asso_hbm,
              key_out, idx_out, cnt_out, psum_out, pl_out,
              beta_buf, p_buf, asso_buf, plo_buf,
              key_tab, idx_tab, cnt_tab, psum_tab,
              lkey, lidx, lcnt, lpsum,
              mkey, midx, mcnt, mpsum,
              okey, oidx, ocnt, opsum,
              psum_all, cnt_all,
              sh_key, sh_idx, sh_cnt, sh_psum,
              sh_psum_m, sh_cnt_m):
    wid = lax.axis_index("s")
    start = wid * CS
    lane = lax.iota(jnp.int32, 16)

    @pl.when(wid < NSC - 1)
    def _():
        pltpu.sync_copy(beta_hbm.at[pl.ds(start, CS)], beta_buf)
        pltpu.sync_copy(p_hbm.at[pl.ds(start, CS)], p_buf)
        pltpu.sync_copy(asso_hbm.at[pl.ds(start, CS)], asso_buf)

    @pl.when(wid == NSC - 1)
    def _():
        pltpu.sync_copy(beta_hbm.at[pl.ds(CS * (NSC - 1), CS_LAST)],
                        beta_buf.at[pl.ds(0, CS_LAST)])
        pltpu.sync_copy(p_hbm.at[pl.ds(CS * (NSC - 1), CS_LAST)],
                        p_buf.at[pl.ds(0, CS_LAST)])
        pltpu.sync_copy(asso_hbm.at[pl.ds(CS * (NSC - 1), CS_LAST)],
                        asso_buf.at[pl.ds(0, CS_LAST)])

    def init_body(i, _):
        sl = pl.ds(i * 16, 16)
        key_tab[sl] = jnp.full((16,), -1.0, jnp.float32)
        idx_tab[sl] = jnp.zeros((16,), jnp.int32)
        cnt_tab[sl] = jnp.zeros((16,), jnp.float32)
        psum_tab[sl] = jnp.zeros((16,), jnp.float32)
        return 0

    lax.fori_loop(0, K, init_body, 0, unroll=8)

    def scat_body(it, _):
        sl = pl.ds(it * 16, 16)
        bv = jnp.clip(beta_buf[sl], 0.0, BMAX)
        pv = p_buf[sl]
        av = asso_buf[sl]
        gidx = start + it * 16 + lane
        addr = lane * K + av
        cur = plsc.load_gather(key_tab, [addr])
        m = bv > cur
        plsc.store_scatter(key_tab, [addr], bv, mask=m)
        plsc.store_scatter(idx_tab, [addr], gidx, mask=m)
        plsc.addupdate_scatter(cnt_tab, [addr], jnp.ones((16,), jnp.float32))
        plsc.addupdate_scatter(psum_tab, [addr],
                               jnp.maximum(pv, 1e-6))
        return 0

    lax.fori_loop(0, NV_LAST, scat_body, 0, unroll=4)

    @pl.when(wid < NSC - 1)
    def _():
        lax.fori_loop(NV_LAST, NV, scat_body, 0, unroll=4)

    def red_body(g, _):
        k0 = g * 16
        mx = jnp.full((16,), -1.0, jnp.float32)
        sc = jnp.zeros((16,), jnp.float32)
        sp = jnp.zeros((16,), jnp.float32)
        cks = []
        for l in range(16):
            sl = pl.ds(k0 + l * K, 16)
            ck = key_tab[sl]
            cks.append(ck)
            mx = jnp.maximum(mx, ck)
            sc = sc + cnt_tab[sl]
            sp = sp + psum_tab[sl]
        mn = jnp.full((16,), BIGI, jnp.int32)
        for l in range(16):
            ci = idx_tab[pl.ds(k0 + l * K, 16)]
            mn = jnp.minimum(mn, jnp.where(cks[l] == mx, ci, BIGI))
        osl = pl.ds(k0, 16)
        lkey[osl] = mx
        lidx[osl] = mn
        lcnt[osl] = sc
        lpsum[osl] = sp
        return 0

    lax.fori_loop(0, K // 16, red_body, 0)

    pltpu.sync_copy(lkey, sh_key.at[pl.ds(wid * K, K)])
    pltpu.sync_copy(lidx, sh_idx.at[pl.ds(wid * K, K)])
    pltpu.sync_copy(lcnt, sh_cnt.at[pl.ds(wid * K, K)])
    pltpu.sync_copy(lpsum, sh_psum.at[pl.ds(wid * K, K)])
    plsc.subcore_barrier()

    k0 = wid * KS
    pltpu.sync_copy(sh_key, mkey)
    pltpu.sync_copy(sh_idx, midx)
    pltpu.sync_copy(sh_cnt, mcnt)
    pltpu.sync_copy(sh_psum, mpsum)

    mx = jnp.full((KS,), -1.0, jnp.float32)
    sc = jnp.zeros((KS,), jnp.float32)
    sp = jnp.zeros((KS,), jnp.float32)
    rks = []
    for w in range(NSC):
        sl = pl.ds(w * K + k0, KS)
        rk = mkey[sl]
        rks.append(rk)
        mx = jnp.maximum(mx, rk)
        sc = sc + mcnt[sl]
        sp = sp + mpsum[sl]
    mn = jnp.full((KS,), BIGI, jnp.int32)
    for w in range(NSC):
        mn = jnp.minimum(
            mn, jnp.where(rks[w] == mx, midx[pl.ds(w * K + k0, KS)], BIGI))
    okey[...] = mx
    oidx[...] = mn
    ocnt[...] = sc
    opsum[...] = sp

    pltpu.sync_copy(okey, key_out.at[pl.ds(k0, KS)])
    pltpu.sync_copy(oidx, idx_out.at[pl.ds(k0, KS)])
    pltpu.sync_copy(ocnt, cnt_out.at[pl.ds(k0, KS)])
    pltpu.sync_copy(opsum, psum_out.at[pl.ds(k0, KS)])
    pltpu.sync_copy(opsum, sh_psum_m.at[pl.ds(k0, KS)])
    pltpu.sync_copy(ocnt, sh_cnt_m.at[pl.ds(k0, KS)])
    plsc.subcore_barrier()

    pltpu.sync_copy(sh_psum_m, psum_all)
    pltpu.sync_copy(sh_cnt_m, cnt_all)

    kexist = jnp.float32(0.0)
    for i in range(K // 16):
        cv = cnt_all[pl.ds(i * 16, 16)]
        kexist = kexist + jnp.sum(jnp.where(cv > 0.0, 1.0, 0.0))
    ktot = jnp.maximum(kexist, 1.0)

    def pls_body(it, _):
        sl = pl.ds(it * 16, 16)
        av = asso_buf[sl]
        ps = plsc.load_gather(psum_all, [av])
        pm = jnp.maximum(p_buf[sl], 1e-6)
        plo_buf[sl] = pm / (jnp.maximum(ps, 1e-9) * ktot)
        return 0

    lax.fori_loop(0, NV_LAST, pls_body, 0, unroll=4)

    @pl.when(wid < NSC - 1)
    def _():
        lax.fori_loop(NV_LAST, NV, pls_body, 0, unroll=4)

    @pl.when(wid < NSC - 1)
    def _():
        pltpu.sync_copy(plo_buf, pl_out.at[pl.ds(start, CS)])

    @pl.when(wid == NSC - 1)
    def _():
        pltpu.sync_copy(plo_buf.at[pl.ds(0, CS_LAST)],
                        pl_out.at[pl.ds(CS * (NSC - 1), CS_LAST)])


def _seg_call(beta_r, p_r, asso_r):
    mesh = plsc.VectorSubcoreMesh(
        core_axis_name="c", subcore_axis_name="s", num_cores=1)
    f = pl.kernel(
        _seg_body,
        out_type=(
            jax.ShapeDtypeStruct((K,), jnp.float32),
            jax.ShapeDtypeStruct((K,), jnp.int32),
            jax.ShapeDtypeStruct((K,), jnp.float32),
            jax.ShapeDtypeStruct((K,), jnp.float32),
            jax.ShapeDtypeStruct((N,), jnp.float32),
        ),
        mesh=mesh,
        compiler_params=pltpu.CompilerParams(needs_layout_passes=False),
        scratch_types=[
            pltpu.VMEM((CS,), jnp.float32),
            pltpu.VMEM((CS,), jnp.float32),
            pltpu.VMEM((CS,), jnp.int32),
            pltpu.VMEM((CS,), jnp.float32),
            pltpu.VMEM((K * 16,), jnp.float32),
            pltpu.VMEM((K * 16,), jnp.int32),
            pltpu.VMEM((K * 16,), jnp.float32),
            pltpu.VMEM((K * 16,), jnp.float32),
            pltpu.VMEM((K,), jnp.float32),
            pltpu.VMEM((K,), jnp.int32),
            pltpu.VMEM((K,), jnp.float32),
            pltpu.VMEM((K,), jnp.float32),
            pltpu.VMEM((NSC * K,), jnp.float32),
            pltpu.VMEM((NSC * K,), jnp.int32),
            pltpu.VMEM((NSC * K,), jnp.float32),
            pltpu.VMEM((NSC * K,), jnp.float32),
            pltpu.VMEM((KS,), jnp.float32),
            pltpu.VMEM((KS,), jnp.int32),
            pltpu.VMEM((KS,), jnp.float32),
            pltpu.VMEM((KS,), jnp.float32),
            pltpu.VMEM((K,), jnp.float32),
            pltpu.VMEM((K,), jnp.float32),
            pltpu.VMEM_SHARED((NSC * K,), jnp.float32),
            pltpu.VMEM_SHARED((NSC * K,), jnp.int32),
            pltpu.VMEM_SHARED((NSC * K,), jnp.float32),
            pltpu.VMEM_SHARED((NSC * K,), jnp.float32),
            pltpu.VMEM_SHARED((K,), jnp.float32),
            pltpu.VMEM_SHARED((K,), jnp.float32),
        ],
    )
    return f(beta_r, p_r, asso_r)


ROWS = 16
NDB = NPAD // (128 * ROWS)


def _dense_body(coordst_ref, q_ref, asso_ref, xa_ref, balpha_ref, cnt_ref,
                lv_ref, lr_ref, lb_ref, pp_ref,
                ca_s, cr_s, amat_s, acc_a, acc_r):
    i = pl.program_id(0)

    @pl.when(i == 0)
    def _():
        ba = balpha_ref[...]
        cnt = cnt_ref[...]
        x = ba * (1.0 - EPS)
        at = 0.5 * jnp.log((1.0 + x) / (1.0 - x))
        qa = at * at + Q_MIN
        exists = jnp.where(cnt > 0.0, 1.0, 0.0)
        ktot = jnp.maximum(jnp.sum(exists), 1.0)
        nk = jnp.maximum(cnt, 1.0)
        nrep = jnp.maximum(jnp.float32(N) - cnt, 1.0)
        ca_s[...] = qa * exists / (nk * ktot)
        cr_s[...] = qa * exists / (nrep * ktot)
        xa = xa_ref[...]
        xa2c = jnp.sum(xa * xa, axis=1, keepdims=True)
        amat_s[...] = jnp.concatenate(
            [-2.0 * xa, xa2c, jnp.ones((K, 1), jnp.float32)], axis=1)
        acc_a[...] = jnp.zeros((1, 128), jnp.float32)
        acc_r[...] = jnp.zeros((1, 128), jnp.float32)

    kio = lax.broadcasted_iota(jnp.int32, (K, 128), 0).astype(jnp.float32)
    ca = ca_s[...]
    cr = cr_s[...]
    amat = amat_s[...]
    qb = q_ref[...].reshape(ROWS, 128)
    ab = asso_ref[...].reshape(ROWS, 128)

    cb = coordst_ref[...]
    x2row = jnp.sum(cb * cb, axis=0, keepdims=True)
    xmat = jnp.concatenate(
        [cb, jnp.ones((1, ROWS * 128), jnp.float32), x2row], axis=0)

    aa = acc_a[...]
    rr = acc_r[...]
    for r in range(ROWS):
        xr = xmat[:, r * 128:(r + 1) * 128]
        d2 = jnp.maximum(
            jnp.dot(amat, xr, preferred_element_type=jnp.float32), 0.0)
        z = d2 + 1e-9
        hinge = jnp.maximum(0.0, 1.0 - z * lax.rsqrt(z))

        asso_row = ab[r:r + 1, :]
        q_row = qb[r:r + 1, :]
        member = asso_row == kio
        attr = jnp.where(member, d2, 0.0)
        rep = jnp.where(member, 0.0, hinge)

        rows_a = jnp.dot(ca, attr, preferred_element_type=jnp.float32)
        rows_r = jnp.dot(cr, rep, preferred_element_type=jnp.float32)
        valid = asso_row >= 0.0
        va = jnp.where(valid, q_row * rows_a, 0.0)
        vr = jnp.where(valid, q_row * rows_r, 0.0)
        pp_ref[:, r:r + 1, :] = (va + vr).reshape(1, 1, 128)
        aa = aa + va
        rr = rr + vr
    acc_a[...] = aa
    acc_r[...] = rr

    @pl.when(i == NDB - 1)
    def _():
        lv_ref[...] = jnp.full((1, 128), jnp.sum(aa), jnp.float32)
        lr_ref[...] = jnp.full((1, 128), jnp.sum(rr), jnp.float32)
        cnt = cnt_ref[...]
        exists = jnp.where(cnt > 0.0, 1.0, 0.0)
        ktot = jnp.maximum(jnp.sum(exists), 1.0)
        lb = jnp.sum((1.0 - balpha_ref[...]) * exists) / ktot
        lb_ref[...] = jnp.full((1, 128), lb, jnp.float32)


def _dense_call(coords_t, q_pad, asso_fp, x_alpha, balpha2, cnt2):
    return pl.pallas_call(
        _dense_body,
        grid=(NDB,),
        in_specs=[
            pl.BlockSpec((C, ROWS * 128), lambda i: (0, i)),
            pl.BlockSpec((1, ROWS, 128), lambda i: (i, 0, 0)),
            pl.BlockSpec((1, ROWS, 128), lambda i: (i, 0, 0)),
            pl.BlockSpec((K, C), lambda i: (0, 0)),
            pl.BlockSpec((1, K), lambda i: (0, 0)),
            pl.BlockSpec((1, K), lambda i: (0, 0)),
        ],
        out_specs=[
            pl.BlockSpec((1, 128), lambda i: (0, 0)),
            pl.BlockSpec((1, 128), lambda i: (0, 0)),
            pl.BlockSpec((1, 128), lambda i: (0, 0)),
            pl.BlockSpec((1, ROWS, 128), lambda i: (i, 0, 0)),
        ],
        out_shape=[
            jax.ShapeDtypeStruct((1, 128), jnp.float32),
            jax.ShapeDtypeStruct((1, 128), jnp.float32),
            jax.ShapeDtypeStruct((1, 128), jnp.float32),
            jax.ShapeDtypeStruct((NDB, ROWS, 128), jnp.float32),
        ],
        scratch_shapes=[
            pltpu.VMEM((1, K), jnp.float32),
            pltpu.VMEM((1, K), jnp.float32),
            pltpu.VMEM((K, C + 2), jnp.float32),
            pltpu.VMEM((1, 128), jnp.float32),
            pltpu.VMEM((1, 128), jnp.float32),
        ],
    )(coords_t, q_pad, asso_fp, x_alpha, balpha2, cnt2)


@jax.jit
def _run(beta, coords, asso_idx):
    beta_r = beta.reshape(-1)
    beta_pad = jnp.pad(beta_r, (0, NPAD - N)).reshape(NPAD // 128, 128)
    q_pad, p_pad = _qp_call(beta_pad)

    asso_r = asso_idx.reshape(-1)
    asso_fp = jnp.pad(asso_r.astype(jnp.float32), (0, NPAD - N),
                      constant_values=-1.0).reshape(NPAD // 128, 128)
    balpha, aidx, cnt, psum, pl_sc = _seg_call(
        beta_r, p_pad.reshape(-1), asso_r)
    del psum

    x_alpha = coords[aidx]
    coords_t = jnp.pad(coords.T, ((0, 0), (0, NPAD - N)))

    lv, lr, lb, pp_pad = _dense_call(
        coords_t, q_pad.reshape(NDB, ROWS, 128),
        asso_fp.reshape(NDB, ROWS, 128),
        x_alpha, balpha.reshape(1, K), cnt.reshape(1, K))

    per_pt = pp_pad.reshape(-1)[:N].reshape(N, 1)
    return (lv[0, 0], lr[0, 0], lb[0, 0],
            pl_sc.reshape(N, 1), per_pt)


def kernel(beta, coords, asso_idx, row_splits):
    del row_splits
    return _run(beta, coords, asso_idx)


# speedup vs baseline: 1.1477x; 1.0712x over previous
import functools

import jax
import jax.numpy as jnp
from jax import lax
from jax.experimental import pallas as pl
from jax.experimental.pallas import tpu as pltpu
from jax.experimental.pallas import tpu_sc as plsc

N = 40000
C = 3
K = 256
Q_MIN = 0.1
S_B = 1.0
EPS = 0.001
BMAX = 1.0 - 1e-4

NSC = 16
CS = 2560
CS_LAST = N - CS * (NSC - 1)
NV = CS // 16
NV_LAST = CS_LAST // 16
KS = K // NSC
BIGI = 2 ** 30

BN = 2048
NB = 20

NPAD = 40960


def _qp_body(beta_ref, q_ref, p_ref):
    b = jnp.clip(beta_ref[...], 0.0, BMAX)
    x = b * (1.0 - EPS)
    at = 0.5 * jnp.log((1.0 + x) / (1.0 - x))
    p = at * at
    p_ref[...] = p
    q_ref[...] = p + Q_MIN


def _qp_call(beta_pad):
    return pl.pallas_call(
        _qp_body,
        out_shape=(
            jax.ShapeDtypeStruct((NPAD // 128, 128), jnp.float32),
            jax.ShapeDtypeStruct((NPAD // 128, 128), jnp.float32),
        ),
    )(beta_pad)


def _seg_body(beta_hbm, p_hbm, asso_hbm,
              key_out, idx_out, cnt_out, psum_out, pl_out,
              beta_buf, p_buf, asso_buf, plo_buf,
              key_tab, idx_tab, cnt_tab, psum_tab,
              lkey, lidx, lcnt, lpsum,
              mkey, midx, mcnt, mpsum,
              okey, oidx, ocnt, opsum,
              psum_all, cnt_all,
              sh_key, sh_idx, sh_cnt, sh_psum,
              sh_psum_m, sh_cnt_m):
    wid = lax.axis_index("s")
    start = wid * CS
    lane = lax.iota(jnp.int32, 16)

    @pl.when(wid < NSC - 1)
    def _():
        pltpu.sync_copy(beta_hbm.at[pl.ds(start, CS)], beta_buf)
        pltpu.sync_copy(p_hbm.at[pl.ds(start, CS)], p_buf)
        pltpu.sync_copy(asso_hbm.at[pl.ds(start, CS)], asso_buf)

    @pl.when(wid == NSC - 1)
    def _():
        pltpu.sync_copy(beta_hbm.at[pl.ds(CS * (NSC - 1), CS_LAST)],
                        beta_buf.at[pl.ds(0, CS_LAST)])
        pltpu.sync_copy(p_hbm.at[pl.ds(CS * (NSC - 1), CS_LAST)],
                        p_buf.at[pl.ds(0, CS_LAST)])
        pltpu.sync_copy(asso_hbm.at[pl.ds(CS * (NSC - 1), CS_LAST)],
                        asso_buf.at[pl.ds(0, CS_LAST)])

    def init_body(i, _):
        sl = pl.ds(i * 16, 16)
        key_tab[sl] = jnp.full((16,), -1.0, jnp.float32)
        idx_tab[sl] = jnp.zeros((16,), jnp.int32)
        cnt_tab[sl] = jnp.zeros((16,), jnp.float32)
        psum_tab[sl] = jnp.zeros((16,), jnp.float32)
        return 0

    lax.fori_loop(0, K, init_body, 0, unroll=8)

    def scat_body(it, _):
        sl = pl.ds(it * 16, 16)
        bv = jnp.clip(beta_buf[sl], 0.0, BMAX)
        pv = p_buf[sl]
        av = asso_buf[sl]
        gidx = start + it * 16 + lane
        addr = lane * K + av
        cur = plsc.load_gather(key_tab, [addr])
        m = bv > cur
        plsc.store_scatter(key_tab, [addr], bv, mask=m)
        plsc.store_scatter(idx_tab, [addr], gidx, mask=m)
        plsc.addupdate_scatter(cnt_tab, [addr], jnp.ones((16,), jnp.float32))
        plsc.addupdate_scatter(psum_tab, [addr],
                               jnp.maximum(pv, 1e-6))
        return 0

    lax.fori_loop(0, NV_LAST, scat_body, 0, unroll=4)

    @pl.when(wid < NSC - 1)
    def _():
        lax.fori_loop(NV_LAST, NV, scat_body, 0, unroll=4)

    def red_body(g, _):
        k0 = g * 16
        mx = jnp.full((16,), -1.0, jnp.float32)
        sc = jnp.zeros((16,), jnp.float32)
        sp = jnp.zeros((16,), jnp.float32)
        cks = []
        for l in range(16):
            sl = pl.ds(k0 + l * K, 16)
            ck = key_tab[sl]
            cks.append(ck)
            mx = jnp.maximum(mx, ck)
            sc = sc + cnt_tab[sl]
            sp = sp + psum_tab[sl]
        mn = jnp.full((16,), BIGI, jnp.int32)
        for l in range(16):
            ci = idx_tab[pl.ds(k0 + l * K, 16)]
            mn = jnp.minimum(mn, jnp.where(cks[l] == mx, ci, BIGI))
        osl = pl.ds(k0, 16)
        lkey[osl] = mx
        lidx[osl] = mn
        lcnt[osl] = sc
        lpsum[osl] = sp
        return 0

    lax.fori_loop(0, K // 16, red_body, 0)

    pltpu.sync_copy(lkey, sh_key.at[pl.ds(wid * K, K)])
    pltpu.sync_copy(lidx, sh_idx.at[pl.ds(wid * K, K)])
    pltpu.sync_copy(lcnt, sh_cnt.at[pl.ds(wid * K, K)])
    pltpu.sync_copy(lpsum, sh_psum.at[pl.ds(wid * K, K)])
    plsc.subcore_barrier()

    k0 = wid * KS
    pltpu.sync_copy(sh_key, mkey)
    pltpu.sync_copy(sh_idx, midx)
    pltpu.sync_copy(sh_cnt, mcnt)
    pltpu.sync_copy(sh_psum, mpsum)

    mx = jnp.full((KS,), -1.0, jnp.float32)
    sc = jnp.zeros((KS,), jnp.float32)
    sp = jnp.zeros((KS,), jnp.float32)
    rks = []
    for w in range(NSC):
        sl = pl.ds(w * K + k0, KS)
        rk = mkey[sl]
        rks.append(rk)
        mx = jnp.maximum(mx, rk)
        sc = sc + mcnt[sl]
        sp = sp + mpsum[sl]
    mn = jnp.full((KS,), BIGI, jnp.int32)
    for w in range(NSC):
        mn = jnp.minimum(
            mn, jnp.where(rks[w] == mx, midx[pl.ds(w * K + k0, KS)], BIGI))
    okey[...] = mx
    oidx[...] = mn
    ocnt[...] = sc
    opsum[...] = sp

    pltpu.sync_copy(okey, key_out.at[pl.ds(k0, KS)])
    pltpu.sync_copy(oidx, idx_out.at[pl.ds(k0, KS)])
    pltpu.sync_copy(ocnt, cnt_out.at[pl.ds(k0, KS)])
    pltpu.sync_copy(opsum, psum_out.at[pl.ds(k0, KS)])
    pltpu.sync_copy(opsum, sh_psum_m.at[pl.ds(k0, KS)])
    pltpu.sync_copy(ocnt, sh_cnt_m.at[pl.ds(k0, KS)])
    plsc.subcore_barrier()

    pltpu.sync_copy(sh_psum_m, psum_all)
    pltpu.sync_copy(sh_cnt_m, cnt_all)

    kexist = jnp.float32(0.0)
    for i in range(K // 16):
        cv = cnt_all[pl.ds(i * 16, 16)]
        kexist = kexist + jnp.sum(jnp.where(cv > 0.0, 1.0, 0.0))
    ktot = jnp.maximum(kexist, 1.0)

    def pls_body(it, _):
        sl = pl.ds(it * 16, 16)
        av = asso_buf[sl]
        ps = plsc.load_gather(psum_all, [av])
        pm = jnp.maximum(p_buf[sl], 1e-6)
        plo_buf[sl] = pm / (jnp.maximum(ps, 1e-9) * ktot)
        return 0

    lax.fori_loop(0, NV_LAST, pls_body, 0, unroll=4)

    @pl.when(wid < NSC - 1)
    def _():
        lax.fori_loop(NV_LAST, NV, pls_body, 0, unroll=4)

    @pl.when(wid < NSC - 1)
    def _():
        pltpu.sync_copy(plo_buf, pl_out.at[pl.ds(start, CS)])

    @pl.when(wid == NSC - 1)
    def _():
        pltpu.sync_copy(plo_buf.at[pl.ds(0, CS_LAST)],
                        pl_out.at[pl.ds(CS * (NSC - 1), CS_LAST)])


def _seg_call(beta_r, p_r, asso_r):
    mesh = plsc.VectorSubcoreMesh(
        core_axis_name="c", subcore_axis_name="s", num_cores=1)
    f = pl.kernel(
        _seg_body,
        out_type=(
            jax.ShapeDtypeStruct((K,), jnp.float32),
            jax.ShapeDtypeStruct((K,), jnp.int32),
            jax.ShapeDtypeStruct((K,), jnp.float32),
            jax.ShapeDtypeStruct((K,), jnp.float32),
            jax.ShapeDtypeStruct((N,), jnp.float32),
        ),
        mesh=mesh,
        compiler_params=pltpu.CompilerParams(needs_layout_passes=False),
        scratch_types=[
            pltpu.VMEM((CS,), jnp.float32),
            pltpu.VMEM((CS,), jnp.float32),
            pltpu.VMEM((CS,), jnp.int32),
            pltpu.VMEM((CS,), jnp.float32),
            pltpu.VMEM((K * 16,), jnp.float32),
            pltpu.VMEM((K * 16,), jnp.int32),
            pltpu.VMEM((K * 16,), jnp.float32),
            pltpu.VMEM((K * 16,), jnp.float32),
            pltpu.VMEM((K,), jnp.float32),
            pltpu.VMEM((K,), jnp.int32),
            pltpu.VMEM((K,), jnp.float32),
            pltpu.VMEM((K,), jnp.float32),
            pltpu.VMEM((NSC * K,), jnp.float32),
            pltpu.VMEM((NSC * K,), jnp.int32),
            pltpu.VMEM((NSC * K,), jnp.float32),
            pltpu.VMEM((NSC * K,), jnp.float32),
            pltpu.VMEM((KS,), jnp.float32),
            pltpu.VMEM((KS,), jnp.int32),
            pltpu.VMEM((KS,), jnp.float32),
            pltpu.VMEM((KS,), jnp.float32),
            pltpu.VMEM((K,), jnp.float32),
            pltpu.VMEM((K,), jnp.float32),
            pltpu.VMEM_SHARED((NSC * K,), jnp.float32),
            pltpu.VMEM_SHARED((NSC * K,), jnp.int32),
            pltpu.VMEM_SHARED((NSC * K,), jnp.float32),
            pltpu.VMEM_SHARED((NSC * K,), jnp.float32),
            pltpu.VMEM_SHARED((K,), jnp.float32),
            pltpu.VMEM_SHARED((K,), jnp.float32),
        ],
    )
    return f(beta_r, p_r, asso_r)


ROWS = 320
NDB = NPAD // (128 * ROWS)


def _dense_body(coordst_ref, q_ref, asso_ref, xa_ref, balpha_ref, cnt_ref,
                lv_ref, lr_ref, lb_ref, pp_ref,
                ca_s, cr_s, amat_s, acc_a, acc_r):
    i = pl.program_id(0)

    @pl.when(i == 0)
    def _():
        ba = balpha_ref[...]
        cnt = cnt_ref[...]
        x = ba * (1.0 - EPS)
        at = 0.5 * jnp.log((1.0 + x) / (1.0 - x))
        qa = at * at + Q_MIN
        exists = jnp.where(cnt > 0.0, 1.0, 0.0)
        ktot = jnp.maximum(jnp.sum(exists), 1.0)
        nk = jnp.maximum(cnt, 1.0)
        nrep = jnp.maximum(jnp.float32(N) - cnt, 1.0)
        ca_s[...] = qa * exists / (nk * ktot)
        cr_s[...] = qa * exists / (nrep * ktot)
        xa = xa_ref[...]
        xa2c = jnp.sum(xa * xa, axis=1, keepdims=True)
        amat_s[...] = jnp.concatenate(
            [-2.0 * xa, xa2c, jnp.ones((K, 1), jnp.float32)], axis=1)
        acc_a[...] = jnp.zeros((1, 128), jnp.float32)
        acc_r[...] = jnp.zeros((1, 128), jnp.float32)

    kio = lax.broadcasted_iota(jnp.int32, (K, 128), 0).astype(jnp.float32)
    ca = ca_s[...]
    cr = cr_s[...]
    amat = amat_s[...]
    qb = q_ref[...].reshape(ROWS, 128)
    ab = asso_ref[...].reshape(ROWS, 128)

    cb = coordst_ref[...]
    x2row = jnp.sum(cb * cb, axis=0, keepdims=True)
    xmat = jnp.concatenate(
        [cb, jnp.ones((1, ROWS * 128), jnp.float32), x2row], axis=0)

    aa = acc_a[...]
    rr = acc_r[...]
    for r in range(ROWS):
        xr = xmat[:, r * 128:(r + 1) * 128]
        d2 = jnp.maximum(
            jnp.dot(amat, xr, preferred_element_type=jnp.float32), 0.0)
        z = d2 + 1e-9
        hinge = jnp.maximum(0.0, 1.0 - z * lax.rsqrt(z))

        asso_row = ab[r:r + 1, :]
        q_row = qb[r:r + 1, :]
        member = asso_row == kio
        attr = jnp.where(member, d2, 0.0)
        rep = jnp.where(member, 0.0, hinge)

        rows_a = jnp.dot(ca, attr, preferred_element_type=jnp.float32)
        rows_r = jnp.dot(cr, rep, preferred_element_type=jnp.float32)
        valid = asso_row >= 0.0
        va = jnp.where(valid, q_row * rows_a, 0.0)
        vr = jnp.where(valid, q_row * rows_r, 0.0)
        pp_ref[:, r:r + 1, :] = (va + vr).reshape(1, 1, 128)
        aa = aa + va
        rr = rr + vr
    acc_a[...] = aa
    acc_r[...] = rr

    @pl.when(i == NDB - 1)
    def _():
        lv_ref[...] = jnp.full((1, 128), jnp.sum(aa), jnp.float32)
        lr_ref[...] = jnp.full((1, 128), jnp.sum(rr), jnp.float32)
        cnt = cnt_ref[...]
        exists = jnp.where(cnt > 0.0, 1.0, 0.0)
        ktot = jnp.maximum(jnp.sum(exists), 1.0)
        lb = jnp.sum((1.0 - balpha_ref[...]) * exists) / ktot
        lb_ref[...] = jnp.full((1, 128), lb, jnp.float32)


def _dense_call(coords_t, q_pad, asso_fp, x_alpha, balpha2, cnt2):
    return pl.pallas_call(
        _dense_body,
        grid=(NDB,),
        in_specs=[
            pl.BlockSpec((C, ROWS * 128), lambda i: (0, i)),
            pl.BlockSpec((1, ROWS, 128), lambda i: (i, 0, 0)),
            pl.BlockSpec((1, ROWS, 128), lambda i: (i, 0, 0)),
            pl.BlockSpec((K, C), lambda i: (0, 0)),
            pl.BlockSpec((1, K), lambda i: (0, 0)),
            pl.BlockSpec((1, K), lambda i: (0, 0)),
        ],
        out_specs=[
            pl.BlockSpec((1, 128), lambda i: (0, 0)),
            pl.BlockSpec((1, 128), lambda i: (0, 0)),
            pl.BlockSpec((1, 128), lambda i: (0, 0)),
            pl.BlockSpec((1, ROWS, 128), lambda i: (i, 0, 0)),
        ],
        out_shape=[
            jax.ShapeDtypeStruct((1, 128), jnp.float32),
            jax.ShapeDtypeStruct((1, 128), jnp.float32),
            jax.ShapeDtypeStruct((1, 128), jnp.float32),
            jax.ShapeDtypeStruct((NDB, ROWS, 128), jnp.float32),
        ],
        scratch_shapes=[
            pltpu.VMEM((1, K), jnp.float32),
            pltpu.VMEM((1, K), jnp.float32),
            pltpu.VMEM((K, C + 2), jnp.float32),
            pltpu.VMEM((1, 128), jnp.float32),
            pltpu.VMEM((1, 128), jnp.float32),
        ],
    )(coords_t, q_pad, asso_fp, x_alpha, balpha2, cnt2)


@jax.jit
def _run(beta, coords, asso_idx):
    beta_r = beta.reshape(-1)
    beta_pad = jnp.pad(beta_r, (0, NPAD - N)).reshape(NPAD // 128, 128)
    q_pad, p_pad = _qp_call(beta_pad)

    asso_r = asso_idx.reshape(-1)
    asso_fp = jnp.pad(asso_r.astype(jnp.float32), (0, NPAD - N),
                      constant_values=-1.0).reshape(NPAD // 128, 128)
    balpha, aidx, cnt, psum, pl_sc = _seg_call(
        beta_r, p_pad.reshape(-1), asso_r)
    del psum

    x_alpha = coords[aidx]
    coords_t = jnp.pad(coords.T, ((0, 0), (0, NPAD - N)))

    lv, lr, lb, pp_pad = _dense_call(
        coords_t, q_pad.reshape(NDB, ROWS, 128),
        asso_fp.reshape(NDB, ROWS, 128),
        x_alpha, balpha.reshape(1, K), cnt.reshape(1, K))

    per_pt = pp_pad.reshape(-1)[:N].reshape(N, 1)
    return (lv[0, 0], lr[0, 0], lb[0, 0],
            pl_sc.reshape(N, 1), per_pt)


def kernel(beta, coords, asso_idx, row_splits):
    del row_splits
    return _run(beta, coords, asso_idx)
